# Initial kernel scaffold; baseline (speedup 1.0000x reference)
import functools

import jax
import jax.numpy as jnp
from jax import lax
from jax.experimental import pallas as pl
from jax.experimental.pallas import tpu as pltpu
from jax.experimental.pallas import tpu_sc as plsc

N = 10000
E = 320000
D = 128

NC = 2
NS = 16
L = 16
NW = NC * NS
C = 512
SUB = C // 128
EPW = 10240
T = EPW // C
E_PAD = EPW * NW
ROWS_PER_SUB = N // NS


def _pq_body(h_ref, w_ref, b_ref, out_ref):
    r = lax.dot_general(
        w_ref[...], h_ref[...],
        dimension_numbers=(((1,), (1,)), ((), ())),
        preferred_element_type=jnp.float32,
    )
    is_p_row = lax.broadcasted_iota(jnp.int32, r.shape, 0) == 0
    out_ref[...] = r + jnp.where(is_p_row, b_ref[0], 0.0)


def _pq(h, w2, b_gate):
    return pl.pallas_call(
        _pq_body,
        out_shape=jax.ShapeDtypeStruct((2, N), jnp.float32),
        in_specs=[
            pl.BlockSpec(memory_space=pltpu.VMEM),
            pl.BlockSpec(memory_space=pltpu.VMEM),
            pl.BlockSpec(memory_space=pltpu.SMEM),
        ],
        out_specs=pl.BlockSpec(memory_space=pltpu.VMEM),
    )(h, w2, b_gate)


_SC_MESH = plsc.VectorSubcoreMesh(core_axis_name="c", subcore_axis_name="s")


@functools.partial(
    pl.kernel,
    out_type=jax.ShapeDtypeStruct((NC, N, D), jnp.float32),
    mesh=_SC_MESH,
    scratch_types=[
        pltpu.VMEM((N,), jnp.float32),
        pltpu.VMEM((N,), jnp.float32),
        pltpu.VMEM((N,), jnp.float32),
        pltpu.VMEM((C,), jnp.int32),
        pltpu.VMEM((C,), jnp.int32),
        pltpu.VMEM((SUB, 128), jnp.int32),
        pltpu.VMEM((SUB, 128), jnp.int32),
        pltpu.VMEM((C, D), jnp.float32),
        pltpu.VMEM((C,), jnp.float32),
        pltpu.VMEM_SHARED((N, D), jnp.float32),
        pltpu.SemaphoreType.DMA,
    ],
)
def _sc_edges(h_hbm, srcr_hbm, dstr_hbm, pq_hbm, d_hbm, z0_hbm, zp_hbm,
              p_v, q_v, d_v, src_v, dst_v, srci_v, dsti_v, rows_v, e_v,
              zacc, sem):
    cid = lax.axis_index("c")
    sid = lax.axis_index("s")
    wid = cid * NS + sid

    pltpu.sync_copy(pq_hbm.at[0], p_v)
    pltpu.sync_copy(pq_hbm.at[1], q_v)
    pltpu.sync_copy(d_hbm, d_v)
    pltpu.sync_copy(z0_hbm.at[pl.ds(sid * ROWS_PER_SUB, ROWS_PER_SUB)],
                    zacc.at[pl.ds(sid * ROWS_PER_SUB, ROWS_PER_SUB)])
    plsc.subcore_barrier()

    ebase = wid * EPW

    def chunk(t, carry):
        base = ebase + t * C
        rbase = base // 128
        pltpu.sync_copy(srcr_hbm.at[pl.ds(rbase, SUB)], srci_v)
        pltpu.sync_copy(dstr_hbm.at[pl.ds(rbase, SUB)], dsti_v)
        pltpu.sync_copy(srcr_hbm.at[pl.ds(rbase, SUB)], src_v.reshape(SUB, 128))
        pltpu.sync_copy(dstr_hbm.at[pl.ds(rbase, SUB)], dst_v.reshape(SUB, 128))
        gathers = [
            pltpu.async_copy(h_hbm.at[srci_v.at[j]],
                             rows_v.at[pl.ds(j * 128, 128)], sem)
            for j in range(SUB)
        ]

        def egrp(i, _):
            s16 = src_v[pl.ds(i * L, L)]
            d16 = dst_v[pl.ds(i * L, L)]
            x = plsc.load_gather(p_v, [d16]) + plsc.load_gather(q_v, [s16])
            dd = plsc.load_gather(d_v, [d16]) * plsc.load_gather(d_v, [s16])
            u = jnp.exp(jnp.abs(x) * -2.0)
            th = (1.0 - u) / (1.0 + u)
            th = jnp.where(x < 0.0, -th, th)
            gid = base + i * L + lax.iota(jnp.int32, L)
            e_v[pl.ds(i * L, L)] = jnp.where(gid < E, th * dd, 0.0)
            return 0

        lax.fori_loop(0, C // L, egrp, 0)
        for g in gathers:
            g.wait()

        def scale(i, _):
            bc = jnp.full((L,), e_v[i], jnp.float32)
            for j in range(D // L):
                rows_v[i, pl.ds(j * L, L)] = rows_v[i, pl.ds(j * L, L)] * bc
            return 0

        lax.fori_loop(0, C, scale, 0)

        for j in range(SUB):
            pltpu.sync_copy(rows_v.at[pl.ds(j * 128, 128)],
                            zacc.at[dsti_v.at[j]], add=True)
        return carry

    lax.fori_loop(0, T, chunk, 0)
    plsc.subcore_barrier()
    pltpu.sync_copy(zacc.at[pl.ds(sid * ROWS_PER_SUB, ROWS_PER_SUB)],
                    zp_hbm.at[cid, pl.ds(sid * ROWS_PER_SUB, ROWS_PER_SUB)])


def _add_body(zp_ref, out_ref):
    out_ref[...] = zp_ref[0] + zp_ref[1]


def _combine(zp):
    return pl.pallas_call(
        _add_body,
        grid=(10,),
        out_shape=jax.ShapeDtypeStruct((N, D), jnp.float32),
        in_specs=[pl.BlockSpec((2, N // 10, D), lambda i: (0, i, 0))],
        out_specs=pl.BlockSpec((N // 10, D), lambda i: (i, 0)),
    )(zp)


@jax.jit
def kernel(h, edge_index, d, W_gate, b_gate):
    src = edge_index[0]
    dst = edge_index[1]
    pad = E_PAD - E
    src_p = jnp.concatenate([src, jnp.zeros((pad,), jnp.int32)])
    dst_p = jnp.concatenate([dst, jnp.zeros((pad,), jnp.int32)])
    srcr = src_p.reshape(E_PAD // 128, 128)
    dstr = dst_p.reshape(E_PAD // 128, 128)
    w2 = W_gate.reshape(2, D)
    pq = _pq(h, w2, b_gate)
    z0 = jnp.zeros((N, D), jnp.float32)
    zp = _sc_edges(h, srcr, dstr, pq, d, z0)
    return _combine(zp)


# speedup vs baseline: 9.4827x; 9.4827x over previous
import functools

import jax
import jax.numpy as jnp
from jax import lax
from jax.experimental import pallas as pl
from jax.experimental.pallas import tpu as pltpu
from jax.experimental.pallas import tpu_sc as plsc

N = 10000
E = 320000
D = 128

NC = 2
NS = 16
L = 16
NW = NC * NS
C = 128
EPW = 10240
T = EPW // C
E_PAD = EPW * NW
N_PAD = 10240
ROWS_PER_SUB = N_PAD // NS
NB = 10


def _pq_body(h_ref, w_ref, d_ref, b_ref, p_ref, q_ref, hd_ref):
    r = lax.dot_general(
        w_ref[...], h_ref[...],
        dimension_numbers=(((1,), (1,)), ((), ())),
        preferred_element_type=jnp.float32,
    )
    p_ref[...] = r[0] + b_ref[0]
    q_ref[...] = r[1]
    hd_ref[...] = h_ref[...] * d_ref[...][:, None]


def _pq(h, w2, d, b_gate):
    return pl.pallas_call(
        _pq_body,
        out_shape=[jax.ShapeDtypeStruct((N,), jnp.float32),
                   jax.ShapeDtypeStruct((N,), jnp.float32),
                   jax.ShapeDtypeStruct((N, D), jnp.float32)],
        in_specs=[
            pl.BlockSpec(memory_space=pltpu.VMEM),
            pl.BlockSpec(memory_space=pltpu.VMEM),
            pl.BlockSpec(memory_space=pltpu.VMEM),
            pl.BlockSpec(memory_space=pltpu.SMEM),
        ],
        out_specs=[pl.BlockSpec(memory_space=pltpu.VMEM),
                   pl.BlockSpec(memory_space=pltpu.VMEM),
                   pl.BlockSpec(memory_space=pltpu.VMEM)],
    )(h, w2, d, b_gate)


_SC_MESH = plsc.VectorSubcoreMesh(core_axis_name="c", subcore_axis_name="s")


@functools.partial(
    pl.kernel,
    out_type=jax.ShapeDtypeStruct((NC, N_PAD, D), jnp.float32),
    mesh=_SC_MESH,
    scratch_types=[
        pltpu.VMEM((N,), jnp.float32),
        pltpu.VMEM((N,), jnp.float32),
        pltpu.VMEM((1, C), jnp.int32),
        pltpu.VMEM((1, C), jnp.int32),
        pltpu.VMEM((C, D), jnp.float32),
        pltpu.VMEM((C,), jnp.float32),
        pltpu.VMEM_SHARED((N_PAD, D), jnp.float32),
        pltpu.SemaphoreType.DMA,
    ],
    compiler_params=pltpu.CompilerParams(use_tc_tiling_on_sc=False,
                                         needs_layout_passes=False),
)
def _sc_edges(hd_hbm, srcr_hbm, dstr_hbm, p_hbm, q_hbm, z0_hbm, zp_hbm,
              p_v, q_v, srci_v, dsti_v, rows_v, e_v, zacc, sem):
    cid = lax.axis_index("c")
    sid = lax.axis_index("s")
    wid = cid * NS + sid

    pltpu.sync_copy(p_hbm, p_v)
    pltpu.sync_copy(q_hbm, q_v)
    pltpu.sync_copy(z0_hbm.at[pl.ds(sid * ROWS_PER_SUB, ROWS_PER_SUB)],
                    zacc.at[pl.ds(sid * ROWS_PER_SUB, ROWS_PER_SUB)])
    plsc.subcore_barrier()

    ebase = wid * EPW

    def chunk(t, carry):
        base = ebase + t * C
        rbase = base // C
        pltpu.sync_copy(srcr_hbm.at[pl.ds(rbase, 1)], srci_v)
        pltpu.sync_copy(dstr_hbm.at[pl.ds(rbase, 1)], dsti_v)
        gat = pltpu.async_copy(hd_hbm.at[srci_v.at[0]], rows_v, sem)

        def egrp(i, _):
            s16 = srci_v[0, pl.ds(i * L, L)]
            d16 = dsti_v[0, pl.ds(i * L, L)]
            x = plsc.load_gather(p_v, [d16]) + plsc.load_gather(q_v, [s16])
            u = jnp.exp(jnp.abs(x) * -2.0)
            th = (1.0 - u) / (1.0 + u)
            th = jnp.where(x < 0.0, -th, th)
            gid = base + i * L + lax.iota(jnp.int32, L)
            e_v[pl.ds(i * L, L)] = jnp.where(gid < E, th, 0.0)
            return 0

        lax.fori_loop(0, C // L, egrp, 0)
        gat.wait()

        def scale(i, _):
            bc = plsc.load_gather(e_v, [jnp.full((L,), i, jnp.int32)])
            for j in range(D // L):
                rows_v[i, pl.ds(j * L, L)] = rows_v[i, pl.ds(j * L, L)] * bc
            return 0

        lax.fori_loop(0, C, scale, 0)

        pltpu.sync_copy(rows_v, zacc.at[dsti_v.at[0]], add=True)
        return carry

    lax.fori_loop(0, T, chunk, 0)
    plsc.subcore_barrier()
    pltpu.sync_copy(zacc.at[pl.ds(sid * ROWS_PER_SUB, ROWS_PER_SUB)],
                    zp_hbm.at[cid, pl.ds(sid * ROWS_PER_SUB, ROWS_PER_SUB)])


def _add_body(zp_ref, d_ref, out_ref):
    out_ref[...] = (zp_ref[0] + zp_ref[1]) * d_ref[0, 0][:, None]


def _combine(zp, d2):
    zsum = pl.pallas_call(
        _add_body,
        grid=(NB,),
        out_shape=jax.ShapeDtypeStruct((N_PAD, D), jnp.float32),
        in_specs=[pl.BlockSpec((2, N_PAD // NB, D), lambda i: (0, i, 0)),
                  pl.BlockSpec((1, 1, N_PAD // NB), lambda i: (i, 0, 0))],
        out_specs=pl.BlockSpec((N_PAD // NB, D), lambda i: (i, 0)),
    )(zp, d2)
    return zsum[:N]


@jax.jit
def kernel(h, edge_index, d, W_gate, b_gate):
    src = edge_index[0]
    dst = edge_index[1]
    pad = E_PAD - E
    src_p = jnp.concatenate([src, jnp.zeros((pad,), jnp.int32)])
    dst_p = jnp.concatenate([dst, jnp.zeros((pad,), jnp.int32)])
    srcr = src_p.reshape(E_PAD // C, C)
    dstr = dst_p.reshape(E_PAD // C, C)
    w2 = W_gate.reshape(2, D)
    p, q, hd = _pq(h, w2, d, b_gate)
    z0 = jnp.zeros((N_PAD, D), jnp.float32)
    zp = _sc_edges(hd, srcr, dstr, p, q, z0)
    d2 = jnp.concatenate([d, jnp.zeros((N_PAD - N,), jnp.float32)]
                         ).reshape(NB, 1, N_PAD // NB)
    return _combine(zp, d2)


# speedup vs baseline: 15.1666x; 1.5994x over previous
import functools

import jax
import jax.numpy as jnp
from jax import lax
from jax.experimental import pallas as pl
from jax.experimental.pallas import tpu as pltpu
from jax.experimental.pallas import tpu_sc as plsc

N = 10000
E = 320000
D = 128

NC = 2
NS = 16
L = 16
NW = NC * NS
C = 96
T = 106
EPW = C * T
E_PAD = EPW * NW
N_PAD = 10240
ROWS_PER_SUB = N_PAD // NS
NB = 10


def _pq_body(h_ref, w_ref, d_ref, b_ref, p_ref, q_ref, hd_ref):
    r = lax.dot_general(
        w_ref[...], h_ref[...],
        dimension_numbers=(((1,), (1,)), ((), ())),
        preferred_element_type=jnp.float32,
    )
    p_ref[...] = r[0] + b_ref[0]
    q_ref[...] = r[1]
    hd_ref[...] = h_ref[...] * d_ref[...][:, None]


def _pq(h, w2, d, b_gate):
    return pl.pallas_call(
        _pq_body,
        out_shape=[jax.ShapeDtypeStruct((N,), jnp.float32),
                   jax.ShapeDtypeStruct((N,), jnp.float32),
                   jax.ShapeDtypeStruct((N, D), jnp.float32)],
        in_specs=[
            pl.BlockSpec(memory_space=pltpu.VMEM),
            pl.BlockSpec(memory_space=pltpu.VMEM),
            pl.BlockSpec(memory_space=pltpu.VMEM),
            pl.BlockSpec(memory_space=pltpu.SMEM),
        ],
        out_specs=[pl.BlockSpec(memory_space=pltpu.VMEM),
                   pl.BlockSpec(memory_space=pltpu.VMEM),
                   pl.BlockSpec(memory_space=pltpu.VMEM)],
    )(h, w2, d, b_gate)


_SC_MESH = plsc.VectorSubcoreMesh(core_axis_name="c", subcore_axis_name="s")


@functools.partial(
    pl.kernel,
    out_type=jax.ShapeDtypeStruct((NC, N_PAD, D), jnp.float32),
    mesh=_SC_MESH,
    scratch_types=[
        pltpu.VMEM((N,), jnp.float32),
        pltpu.VMEM((N,), jnp.float32),
        [pltpu.VMEM((1, C), jnp.int32)] * 2,
        [pltpu.VMEM((1, C), jnp.int32)] * 2,
        [pltpu.VMEM((C, D), jnp.float32)] * 2,
        pltpu.VMEM((C,), jnp.float32),
        pltpu.VMEM_SHARED((N_PAD, D), jnp.float32),
        [pltpu.SemaphoreType.DMA] * 2,
        [pltpu.SemaphoreType.DMA] * 2,
    ],
    compiler_params=pltpu.CompilerParams(use_tc_tiling_on_sc=False,
                                         needs_layout_passes=False),
)
def _sc_edges(hd_hbm, srcr_hbm, dstr_hbm, p_hbm, q_hbm, z0_hbm, zp_hbm,
              p_v, q_v, srci, dsti, rows, e_v, zacc, gsem, ssem):
    cid = lax.axis_index("c")
    sid = lax.axis_index("s")
    wid = cid * NS + sid

    pltpu.sync_copy(p_hbm, p_v)
    pltpu.sync_copy(q_hbm, q_v)
    pltpu.sync_copy(z0_hbm.at[pl.ds(sid * ROWS_PER_SUB, ROWS_PER_SUB)],
                    zacc.at[pl.ds(sid * ROWS_PER_SUB, ROWS_PER_SUB)])
    plsc.subcore_barrier()

    cbase = wid * T

    def stage(ct, b):
        pltpu.sync_copy(srcr_hbm.at[pl.ds(cbase + ct, 1)], srci[b])
        pltpu.sync_copy(dstr_hbm.at[pl.ds(cbase + ct, 1)], dsti[b])
        pltpu.async_copy(hd_hbm.at[srci[b].at[0]], rows[b], gsem[b])

    stage(0, 0)

    def body(tt, carry):
        for b in (0, 1):
            t = 2 * tt + b
            nb = 1 - b

            @pl.when(t + 1 < T)
            def _():
                @pl.when(t >= 1)
                def _():
                    pltpu.make_async_copy(rows[nb], zacc.at[dsti[nb].at[0]],
                                          ssem[nb]).wait()
                stage(t + 1, nb)

            ebase = (cbase + t) * C

            @plsc.parallel_loop(0, C // L, 1, unroll=2)
            def _(i):
                s16 = srci[b][0, pl.ds(i * L, L)]
                d16 = dsti[b][0, pl.ds(i * L, L)]
                x = plsc.load_gather(p_v, [d16]) + plsc.load_gather(q_v, [s16])
                u = jnp.exp(jnp.abs(x) * -2.0)
                th = (1.0 - u) / (1.0 + u)
                th = jnp.where(x < 0.0, -th, th)
                gid = ebase + i * L + lax.iota(jnp.int32, L)
                e_v[pl.ds(i * L, L)] = jnp.where(gid < E, th, 0.0)

            pltpu.make_async_copy(hd_hbm.at[srci[b].at[0]], rows[b],
                                  gsem[b]).wait()

            @plsc.parallel_loop(0, C, 1, unroll=2)
            def _(i):
                bc = plsc.load_gather(e_v, [jnp.full((L,), i, jnp.int32)])
                for j in range(D // L):
                    rows[b][i, pl.ds(j * L, L)] = rows[b][i, pl.ds(j * L, L)] * bc

            pltpu.async_copy(rows[b], zacc.at[dsti[b].at[0]], ssem[b], add=True)
        return carry

    lax.fori_loop(0, T // 2, body, 0)
    for b in (0, 1):
        pltpu.make_async_copy(rows[b], zacc.at[dsti[b].at[0]], ssem[b]).wait()
    plsc.subcore_barrier()
    pltpu.sync_copy(zacc.at[pl.ds(sid * ROWS_PER_SUB, ROWS_PER_SUB)],
                    zp_hbm.at[cid, pl.ds(sid * ROWS_PER_SUB, ROWS_PER_SUB)])


def _add_body(zp_ref, d_ref, out_ref):
    out_ref[...] = (zp_ref[0] + zp_ref[1]) * d_ref[0, 0][:, None]


def _combine(zp, d2):
    zsum = pl.pallas_call(
        _add_body,
        grid=(NB,),
        out_shape=jax.ShapeDtypeStruct((N_PAD, D), jnp.float32),
        in_specs=[pl.BlockSpec((2, N_PAD // NB, D), lambda i: (0, i, 0)),
                  pl.BlockSpec((1, 1, N_PAD // NB), lambda i: (i, 0, 0))],
        out_specs=pl.BlockSpec((N_PAD // NB, D), lambda i: (i, 0)),
    )(zp, d2)
    return zsum[:N]


@jax.jit
def kernel(h, edge_index, d, W_gate, b_gate):
    src = edge_index[0]
    dst = edge_index[1]
    pad = E_PAD - E
    src_p = jnp.concatenate([src, jnp.zeros((pad,), jnp.int32)])
    dst_p = jnp.concatenate([dst, jnp.zeros((pad,), jnp.int32)])
    srcr = src_p.reshape(E_PAD // C, C)
    dstr = dst_p.reshape(E_PAD // C, C)
    w2 = W_gate.reshape(2, D)
    p, q, hd = _pq(h, w2, d, b_gate)
    z0 = jnp.zeros((N_PAD, D), jnp.float32)
    zp = _sc_edges(hd, srcr, dstr, p, q, z0)
    d2 = jnp.concatenate([d, jnp.zeros((N_PAD - N,), jnp.float32)]
                         ).reshape(NB, 1, N_PAD // NB)
    return _combine(zp, d2)
